# baseline (speedup 1.0000x reference)
import jax
import jax.numpy as jnp
from jax.experimental import pallas as pl
from jax.experimental.pallas import tpu as pltpu

F_IN = 142
T_SEG = 32
N_SEG = 4
T_TOT = N_SEG * T_SEG
H = 128
Z = 128
D_OUT = 256
G = 3 * H


def _round_up(x, m):
    return ((x + m - 1) // m) * m


def _gru_kernel(x_ref,
                wih_ref,
                bfold_ref,
                whf_ref,
                whb_ref,
                bhn_ref,
                wtail_ref,
                bpos_ref,
                out_ref,
                gx_ref):
    B, _, F = x_ref.shape
    N = N_SEG * B

    wih = wih_ref[...]
    bfold = bfold_ref[...]
    for s in range(N_SEG):
        for t in range(T_SEG):
            gx_t = jnp.dot(x_ref[:, s * T_SEG + t, :], wih,
                           preferred_element_type=jnp.float32) + bfold
            gx_ref[t, s * B:(s + 1) * B, :] = gx_t.astype(jnp.bfloat16)

    whf = whf_ref[...]
    whb = whb_ref[...]
    bhn_f = bhn_ref[0, 2 * H:G]
    bhn_b = bhn_ref[0, G + 2 * H:]

    def body(t, carry):
        h_f, h_b = carry
        gh_f = jnp.dot(h_f.astype(jnp.bfloat16), whf,
                       preferred_element_type=jnp.float32)
        gh_b = jnp.dot(h_b.astype(jnp.bfloat16), whb,
                       preferred_element_type=jnp.float32)
        gxf = gx_ref[t, :, :G]
        gxb = gx_ref[T_SEG - 1 - t, :, G:]

        r_f = jax.nn.sigmoid(gxf[:, 0:H] + gh_f[:, 0:H])
        z_f = jax.nn.sigmoid(gxf[:, H:2 * H] + gh_f[:, H:2 * H])
        n_f = jnp.tanh(gxf[:, 2 * H:] + r_f * (gh_f[:, 2 * H:] + bhn_f))
        h_f = n_f + z_f * (h_f - n_f)

        r_b = jax.nn.sigmoid(gxb[:, 0:H] + gh_b[:, 0:H])
        z_b = jax.nn.sigmoid(gxb[:, H:2 * H] + gh_b[:, H:2 * H])
        n_b = jnp.tanh(gxb[:, 2 * H:] + r_b * (gh_b[:, 2 * H:] + bhn_b))
        h_b = n_b + z_b * (h_b - n_b)
        return h_f, h_b

    h0 = jnp.zeros((N, H), jnp.float32)
    h_f, h_b = jax.lax.fori_loop(0, T_SEG, body, (h0, h0), unroll=4)

    res = (jnp.dot(h_f, wtail_ref[:H], preferred_element_type=jnp.float32)
           + jnp.dot(h_b, wtail_ref[H:], preferred_element_type=jnp.float32))
    out_ref[...] = res.reshape(N_SEG, B, D_OUT) + bpos_ref[...]


def kernel(mel_pr, w_ih_f, w_hh_f, b_ih_f, b_hh_f, w_ih_b, w_hh_b,
           b_ih_b, b_hh_b, w_mu, b_mu, w_sq, b_sq, pos_tab):
    bs, t_total, f = mel_pr.shape
    assert t_total == T_TOT and f == F_IN

    tile_b = min(128, _round_up(bs, 8))
    bs_pad = _round_up(bs, tile_b)
    x = mel_pr
    if bs_pad != bs:
        x = jnp.pad(x, ((0, bs_pad - bs), (0, 0), (0, 0)))

    bih = jnp.concatenate([b_ih_f, b_ih_b], axis=1)
    bhh = jnp.concatenate([b_hh_f, b_hh_b], axis=1)
    n_cols = jnp.concatenate([jnp.zeros((1, 2 * H)), jnp.ones((1, H)),
                              jnp.zeros((1, 2 * H)), jnp.ones((1, H))], axis=1)
    b_fold = (bih + bhh * (1.0 - n_cols)).astype(jnp.float32)
    bhn = (bhh * n_cols).astype(jnp.float32)

    wih = jnp.concatenate([w_ih_f.T, w_ih_b.T], axis=1)
    whf = w_hh_f.T.astype(jnp.bfloat16)
    whb = w_hh_b.T.astype(jnp.bfloat16)

    wtail = (w_sq @ w_mu[Z:, :]).T
    btail = b_mu[:, Z:] @ w_sq.T + b_sq
    bpos = (btail[:, None, :] + pos_tab[:, None, :])
    bpos = jnp.broadcast_to(bpos, (N_SEG, tile_b, D_OUT))

    grid = (bs_pad // tile_b,)
    n_tile = N_SEG * tile_b

    flops = (2 * T_TOT * bs_pad * F_IN * 2 * G
             + 2 * T_TOT * bs_pad * H * 2 * G
             + 2 * N_SEG * bs_pad * 2 * H * D_OUT)
    transcendentals = T_TOT * bs_pad * 2 * G
    bytes_accessed = 4 * T_TOT * bs_pad * F_IN \
        + 4 * (F_IN * 2 * G + H * 2 * G) \
        + 4 * (2 * H * D_OUT + N_SEG * tile_b * D_OUT + N_SEG * bs_pad * D_OUT)

    out_sm = pl.pallas_call(
        _gru_kernel,
        out_shape=jax.ShapeDtypeStruct((N_SEG, bs_pad, D_OUT), jnp.float32),
        grid=grid,
        in_specs=[
            pl.BlockSpec((tile_b, T_TOT, F_IN), lambda i: (i, 0, 0)),
            pl.BlockSpec((F_IN, 2 * G), lambda i: (0, 0)),
            pl.BlockSpec((1, 2 * G), lambda i: (0, 0)),
            pl.BlockSpec((H, G), lambda i: (0, 0)),
            pl.BlockSpec((H, G), lambda i: (0, 0)),
            pl.BlockSpec((1, 2 * G), lambda i: (0, 0)),
            pl.BlockSpec((2 * H, D_OUT), lambda i: (0, 0)),
            pl.BlockSpec((N_SEG, tile_b, D_OUT), lambda i: (0, 0, 0)),
        ],
        out_specs=pl.BlockSpec((N_SEG, tile_b, D_OUT), lambda i: (0, i, 0)),
        scratch_shapes=[pltpu.VMEM((T_SEG, n_tile, 2 * G), jnp.bfloat16)],
        compiler_params=pltpu.CompilerParams(
            dimension_semantics=("parallel",),
            vmem_limit_bytes=63 * 1024 * 1024,
        ),
        cost_estimate=pl.CostEstimate(flops=flops,
                                      transcendentals=transcendentals,
                                      bytes_accessed=bytes_accessed),
    )(x, wih, b_fold, whf, whb, bhn, wtail, bpos)

    return jnp.transpose(out_sm[:, :bs, :], (1, 0, 2))


# speedup vs baseline: 1.6266x; 1.6266x over previous
import jax
import jax.numpy as jnp
from jax.experimental import pallas as pl
from jax.experimental.pallas import tpu as pltpu

F_IN = 142
T_SEG = 32
N_SEG = 4
T_TOT = N_SEG * T_SEG
H = 128
Z = 128
D_OUT = 256
G = 3 * H


def _round_up(x, m):
    return ((x + m - 1) // m) * m


def _gru_kernel(x_ref,
                wih_ref,
                bfold_ref,
                whf_ref,
                whb_ref,
                bhn_ref,
                wtail_ref,
                bpos_ref,
                out_ref,
                gx_ref):
    F, B, _ = x_ref.shape
    N = N_SEG * B

    xT = jnp.transpose(x_ref[...], (0, 2, 1))
    wih = wih_ref[...]
    bfold = bfold_ref[...]
    dn = (((0,), (0,)), ((), ()))
    for s in range(N_SEG):
        for c in range(T_SEG // 8):
            xc = xT[:, s * T_SEG + 8 * c:s * T_SEG + 8 * c + 8, :]
            gx_c = jax.lax.dot_general(
                xc.reshape(F, 8 * B), wih, dn,
                preferred_element_type=jnp.float32) + bfold
            gx_ref[8 * c:8 * c + 8, s * B:(s + 1) * B, :] = (
                gx_c.reshape(8, B, 2 * G).astype(jnp.bfloat16))

    whf = whf_ref[...]
    whb = whb_ref[...]
    bhn_f = bhn_ref[0, 2 * H:G]
    bhn_b = bhn_ref[0, G + 2 * H:]

    def body(t, carry):
        h_f, h_b = carry
        gh_f = jnp.dot(h_f.astype(jnp.bfloat16), whf,
                       preferred_element_type=jnp.float32)
        gh_b = jnp.dot(h_b.astype(jnp.bfloat16), whb,
                       preferred_element_type=jnp.float32)
        gxf = gx_ref[t, :, :G]
        gxb = gx_ref[T_SEG - 1 - t, :, G:]

        r_f = jax.nn.sigmoid(gxf[:, 0:H] + gh_f[:, 0:H])
        z_f = jax.nn.sigmoid(gxf[:, H:2 * H] + gh_f[:, H:2 * H])
        n_f = jnp.tanh(gxf[:, 2 * H:] + r_f * (gh_f[:, 2 * H:] + bhn_f))
        h_f = n_f + z_f * (h_f - n_f)

        r_b = jax.nn.sigmoid(gxb[:, 0:H] + gh_b[:, 0:H])
        z_b = jax.nn.sigmoid(gxb[:, H:2 * H] + gh_b[:, H:2 * H])
        n_b = jnp.tanh(gxb[:, 2 * H:] + r_b * (gh_b[:, 2 * H:] + bhn_b))
        h_b = n_b + z_b * (h_b - n_b)
        return h_f, h_b

    h0 = jnp.zeros((N, H), jnp.float32)
    h_f, h_b = jax.lax.fori_loop(0, T_SEG, body, (h0, h0), unroll=4)

    res = (jnp.dot(h_f, wtail_ref[:H], preferred_element_type=jnp.float32)
           + jnp.dot(h_b, wtail_ref[H:], preferred_element_type=jnp.float32))
    out_ref[...] = res.reshape(N_SEG, B, D_OUT) + bpos_ref[...]


def kernel(mel_pr, w_ih_f, w_hh_f, b_ih_f, b_hh_f, w_ih_b, w_hh_b,
           b_ih_b, b_hh_b, w_mu, b_mu, w_sq, b_sq, pos_tab):
    bs, t_total, f = mel_pr.shape
    assert t_total == T_TOT and f == F_IN

    tile_b = min(128, _round_up(bs, 8))
    bs_pad = _round_up(bs, tile_b)
    x = jnp.transpose(mel_pr, (2, 0, 1))
    if bs_pad != bs:
        x = jnp.pad(x, ((0, 0), (0, bs_pad - bs), (0, 0)))

    bih = jnp.concatenate([b_ih_f, b_ih_b], axis=1)
    bhh = jnp.concatenate([b_hh_f, b_hh_b], axis=1)
    n_cols = jnp.concatenate([jnp.zeros((1, 2 * H)), jnp.ones((1, H)),
                              jnp.zeros((1, 2 * H)), jnp.ones((1, H))], axis=1)
    b_fold = (bih + bhh * (1.0 - n_cols)).astype(jnp.float32)
    bhn = (bhh * n_cols).astype(jnp.float32)

    wih = jnp.concatenate([w_ih_f.T, w_ih_b.T], axis=1)
    whf = w_hh_f.T.astype(jnp.bfloat16)
    whb = w_hh_b.T.astype(jnp.bfloat16)

    wtail = (w_sq @ w_mu[Z:, :]).T
    btail = b_mu[:, Z:] @ w_sq.T + b_sq
    bpos = (btail[:, None, :] + pos_tab[:, None, :])
    bpos = jnp.broadcast_to(bpos, (N_SEG, tile_b, D_OUT))

    grid = (bs_pad // tile_b,)
    n_tile = N_SEG * tile_b

    flops = (2 * T_TOT * bs_pad * F_IN * 2 * G
             + 2 * T_TOT * bs_pad * H * 2 * G
             + 2 * N_SEG * bs_pad * 2 * H * D_OUT)
    transcendentals = T_TOT * bs_pad * 2 * G
    bytes_accessed = 4 * T_TOT * bs_pad * F_IN \
        + 4 * (F_IN * 2 * G + H * 2 * G) \
        + 4 * (2 * H * D_OUT + N_SEG * tile_b * D_OUT + N_SEG * bs_pad * D_OUT)

    out_sm = pl.pallas_call(
        _gru_kernel,
        out_shape=jax.ShapeDtypeStruct((N_SEG, bs_pad, D_OUT), jnp.float32),
        grid=grid,
        in_specs=[
            pl.BlockSpec((F_IN, tile_b, T_TOT), lambda i: (0, i, 0)),
            pl.BlockSpec((F_IN, 2 * G), lambda i: (0, 0)),
            pl.BlockSpec((1, 2 * G), lambda i: (0, 0)),
            pl.BlockSpec((H, G), lambda i: (0, 0)),
            pl.BlockSpec((H, G), lambda i: (0, 0)),
            pl.BlockSpec((1, 2 * G), lambda i: (0, 0)),
            pl.BlockSpec((2 * H, D_OUT), lambda i: (0, 0)),
            pl.BlockSpec((N_SEG, tile_b, D_OUT), lambda i: (0, 0, 0)),
        ],
        out_specs=pl.BlockSpec((N_SEG, tile_b, D_OUT), lambda i: (0, i, 0)),
        scratch_shapes=[pltpu.VMEM((T_SEG, n_tile, 2 * G), jnp.bfloat16)],
        compiler_params=pltpu.CompilerParams(
            dimension_semantics=("parallel",),
            vmem_limit_bytes=63 * 1024 * 1024,
        ),
        cost_estimate=pl.CostEstimate(flops=flops,
                                      transcendentals=transcendentals,
                                      bytes_accessed=bytes_accessed),
    )(x, wih, b_fold, whf, whb, bhn, wtail, bpos)

    return jnp.transpose(out_sm[:, :bs, :], (1, 0, 2))


# speedup vs baseline: 1.6759x; 1.0303x over previous
import jax
import jax.numpy as jnp
from jax.experimental import pallas as pl
from jax.experimental.pallas import tpu as pltpu

F_IN = 142
T_SEG = 32
N_SEG = 4
T_TOT = N_SEG * T_SEG
H = 128
Z = 128
D_OUT = 256
G = 3 * H


def _round_up(x, m):
    return ((x + m - 1) // m) * m


def _gru_kernel(x_ref,
                wih_ref,
                bfold_ref,
                whf_ref,
                whb_ref,
                bhn_ref,
                wtail_ref,
                bpos_ref,
                out_ref,
                gx_ref):
    F, B, _ = x_ref.shape
    N = N_SEG * B

    xT = jnp.transpose(x_ref[...], (0, 2, 1))
    wih = wih_ref[...]
    bfold = bfold_ref[...]
    dn = (((0,), (0,)), ((), ()))
    for s in range(N_SEG):
        xc = xT[:, s * T_SEG:(s + 1) * T_SEG, :]
        gx_s = jax.lax.dot_general(
            xc.reshape(F, T_SEG * B), wih, dn,
            preferred_element_type=jnp.float32) + bfold
        gx_ref[:, s * B:(s + 1) * B, :] = (
            gx_s.reshape(T_SEG, B, 2 * G).astype(jnp.bfloat16))

    whf = whf_ref[...]
    whb = whb_ref[...]
    bhn_f = bhn_ref[0, 2 * H:G]
    bhn_b = bhn_ref[0, G + 2 * H:]

    def body(t, carry):
        h_f, h_b = carry
        gh_f = jnp.dot(h_f.astype(jnp.bfloat16), whf,
                       preferred_element_type=jnp.float32)
        gh_b = jnp.dot(h_b.astype(jnp.bfloat16), whb,
                       preferred_element_type=jnp.float32)
        gxf = gx_ref[t, :, :G]
        gxb = gx_ref[T_SEG - 1 - t, :, G:]

        r_f = jax.nn.sigmoid(gxf[:, 0:H] + gh_f[:, 0:H])
        z_f = jax.nn.sigmoid(gxf[:, H:2 * H] + gh_f[:, H:2 * H])
        n_f = jnp.tanh(gxf[:, 2 * H:] + r_f * (gh_f[:, 2 * H:] + bhn_f))
        h_f = n_f + z_f * (h_f - n_f)

        r_b = jax.nn.sigmoid(gxb[:, 0:H] + gh_b[:, 0:H])
        z_b = jax.nn.sigmoid(gxb[:, H:2 * H] + gh_b[:, H:2 * H])
        n_b = jnp.tanh(gxb[:, 2 * H:] + r_b * (gh_b[:, 2 * H:] + bhn_b))
        h_b = n_b + z_b * (h_b - n_b)
        return h_f, h_b

    h0 = jnp.zeros((N, H), jnp.float32)
    h_f, h_b = jax.lax.fori_loop(0, T_SEG, body, (h0, h0), unroll=8)

    res = (jnp.dot(h_f, wtail_ref[:H], preferred_element_type=jnp.float32)
           + jnp.dot(h_b, wtail_ref[H:], preferred_element_type=jnp.float32))
    out_ref[...] = res.reshape(N_SEG, B, D_OUT) + bpos_ref[...]


def kernel(mel_pr, w_ih_f, w_hh_f, b_ih_f, b_hh_f, w_ih_b, w_hh_b,
           b_ih_b, b_hh_b, w_mu, b_mu, w_sq, b_sq, pos_tab):
    bs, t_total, f = mel_pr.shape
    assert t_total == T_TOT and f == F_IN

    tile_b = min(128, _round_up(bs, 8))
    bs_pad = _round_up(bs, tile_b)
    x = jnp.transpose(mel_pr, (2, 0, 1))
    if bs_pad != bs:
        x = jnp.pad(x, ((0, 0), (0, bs_pad - bs), (0, 0)))

    bih = jnp.concatenate([b_ih_f, b_ih_b], axis=1)
    bhh = jnp.concatenate([b_hh_f, b_hh_b], axis=1)
    n_cols = jnp.concatenate([jnp.zeros((1, 2 * H)), jnp.ones((1, H)),
                              jnp.zeros((1, 2 * H)), jnp.ones((1, H))], axis=1)
    b_fold = (bih + bhh * (1.0 - n_cols)).astype(jnp.float32)
    bhn = (bhh * n_cols).astype(jnp.float32)

    wih = jnp.concatenate([w_ih_f.T, w_ih_b.T], axis=1)
    whf = w_hh_f.T.astype(jnp.bfloat16)
    whb = w_hh_b.T.astype(jnp.bfloat16)

    wtail = (w_sq @ w_mu[Z:, :]).T
    btail = b_mu[:, Z:] @ w_sq.T + b_sq
    bpos = (btail[:, None, :] + pos_tab[:, None, :])
    bpos = jnp.broadcast_to(bpos, (N_SEG, tile_b, D_OUT))

    grid = (bs_pad // tile_b,)
    n_tile = N_SEG * tile_b

    flops = (2 * T_TOT * bs_pad * F_IN * 2 * G
             + 2 * T_TOT * bs_pad * H * 2 * G
             + 2 * N_SEG * bs_pad * 2 * H * D_OUT)
    transcendentals = T_TOT * bs_pad * 2 * G
    bytes_accessed = 4 * T_TOT * bs_pad * F_IN \
        + 4 * (F_IN * 2 * G + H * 2 * G) \
        + 4 * (2 * H * D_OUT + N_SEG * tile_b * D_OUT + N_SEG * bs_pad * D_OUT)

    out_sm = pl.pallas_call(
        _gru_kernel,
        out_shape=jax.ShapeDtypeStruct((N_SEG, bs_pad, D_OUT), jnp.float32),
        grid=grid,
        in_specs=[
            pl.BlockSpec((F_IN, tile_b, T_TOT), lambda i: (0, i, 0)),
            pl.BlockSpec((F_IN, 2 * G), lambda i: (0, 0)),
            pl.BlockSpec((1, 2 * G), lambda i: (0, 0)),
            pl.BlockSpec((H, G), lambda i: (0, 0)),
            pl.BlockSpec((H, G), lambda i: (0, 0)),
            pl.BlockSpec((1, 2 * G), lambda i: (0, 0)),
            pl.BlockSpec((2 * H, D_OUT), lambda i: (0, 0)),
            pl.BlockSpec((N_SEG, tile_b, D_OUT), lambda i: (0, 0, 0)),
        ],
        out_specs=pl.BlockSpec((N_SEG, tile_b, D_OUT), lambda i: (0, i, 0)),
        scratch_shapes=[pltpu.VMEM((T_SEG, n_tile, 2 * G), jnp.bfloat16)],
        compiler_params=pltpu.CompilerParams(
            dimension_semantics=("parallel",),
            vmem_limit_bytes=63 * 1024 * 1024,
        ),
        cost_estimate=pl.CostEstimate(flops=flops,
                                      transcendentals=transcendentals,
                                      bytes_accessed=bytes_accessed),
    )(x, wih, b_fold, whf, whb, bhn, wtail, bpos)

    return jnp.transpose(out_sm[:, :bs, :], (1, 0, 2))


# speedup vs baseline: 1.7710x; 1.0568x over previous
import jax
import jax.numpy as jnp
from jax.experimental import pallas as pl
from jax.experimental.pallas import tpu as pltpu

F_IN = 142
T_SEG = 32
N_SEG = 4
T_TOT = N_SEG * T_SEG
H = 128
Z = 128
D_OUT = 256
G = 3 * H


def _round_up(x, m):
    return ((x + m - 1) // m) * m


def _gru_kernel(x_ref,
                wih_ref,
                bfold_ref,
                whf_ref,
                whb_ref,
                bhn_ref,
                wtail_ref,
                bpos_ref,
                out_ref,
                gx_ref):
    F, B, _ = x_ref.shape
    N = N_SEG * B

    xT = jnp.transpose(x_ref[...].astype(jnp.bfloat16), (0, 2, 1))
    wih = wih_ref[...]
    bfold = bfold_ref[...]
    dn = (((0,), (0,)), ((), ()))
    for s in range(N_SEG):
        xc = xT[:, s * T_SEG:(s + 1) * T_SEG, :]
        gx_s = jax.lax.dot_general(
            xc.reshape(F, T_SEG * B), wih, dn,
            preferred_element_type=jnp.float32) + bfold
        gx_ref[:, s * B:(s + 1) * B, :] = (
            gx_s.reshape(T_SEG, B, 2 * G).astype(jnp.bfloat16))

    whf = whf_ref[...]
    whb = whb_ref[...]
    bhn_f = bhn_ref[0, 2 * H:G]
    bhn_b = bhn_ref[0, G + 2 * H:]

    def sig(v):
        return 0.5 * jnp.tanh(0.5 * v) + 0.5

    def body(t, carry):
        h_f, h_b = carry
        gh_f = jnp.dot(h_f.astype(jnp.bfloat16), whf,
                       preferred_element_type=jnp.float32)
        gh_b = jnp.dot(h_b.astype(jnp.bfloat16), whb,
                       preferred_element_type=jnp.float32)
        gxf = gx_ref[t, :, :G]
        gxb = gx_ref[T_SEG - 1 - t, :, G:]

        r_f = sig(gxf[:, 0:H] + gh_f[:, 0:H])
        z_f = sig(gxf[:, H:2 * H] + gh_f[:, H:2 * H])
        n_f = jnp.tanh(gxf[:, 2 * H:] + r_f * (gh_f[:, 2 * H:] + bhn_f))
        h_f = n_f + z_f * (h_f - n_f)

        r_b = sig(gxb[:, 0:H] + gh_b[:, 0:H])
        z_b = sig(gxb[:, H:2 * H] + gh_b[:, H:2 * H])
        n_b = jnp.tanh(gxb[:, 2 * H:] + r_b * (gh_b[:, 2 * H:] + bhn_b))
        h_b = n_b + z_b * (h_b - n_b)
        return h_f, h_b

    h0 = jnp.zeros((N, H), jnp.float32)
    h_f, h_b = jax.lax.fori_loop(0, T_SEG, body, (h0, h0), unroll=8)

    res = (jnp.dot(h_f, wtail_ref[:H], preferred_element_type=jnp.float32)
           + jnp.dot(h_b, wtail_ref[H:], preferred_element_type=jnp.float32))
    out_ref[...] = res.reshape(N_SEG, B, D_OUT) + bpos_ref[...]


def kernel(mel_pr, w_ih_f, w_hh_f, b_ih_f, b_hh_f, w_ih_b, w_hh_b,
           b_ih_b, b_hh_b, w_mu, b_mu, w_sq, b_sq, pos_tab):
    bs, t_total, f = mel_pr.shape
    assert t_total == T_TOT and f == F_IN

    tile_b = min(128, _round_up(bs, 8))
    bs_pad = _round_up(bs, tile_b)
    x = jnp.transpose(mel_pr, (2, 0, 1))
    if bs_pad != bs:
        x = jnp.pad(x, ((0, 0), (0, bs_pad - bs), (0, 0)))

    bih = jnp.concatenate([b_ih_f, b_ih_b], axis=1)
    bhh = jnp.concatenate([b_hh_f, b_hh_b], axis=1)
    n_cols = jnp.concatenate([jnp.zeros((1, 2 * H)), jnp.ones((1, H)),
                              jnp.zeros((1, 2 * H)), jnp.ones((1, H))], axis=1)
    b_fold = (bih + bhh * (1.0 - n_cols)).astype(jnp.float32)
    bhn = (bhh * n_cols).astype(jnp.float32)

    wih = jnp.concatenate([w_ih_f.T, w_ih_b.T],
                          axis=1).astype(jnp.bfloat16)
    whf = w_hh_f.T.astype(jnp.bfloat16)
    whb = w_hh_b.T.astype(jnp.bfloat16)

    wtail = (w_sq @ w_mu[Z:, :]).T
    btail = b_mu[:, Z:] @ w_sq.T + b_sq
    bpos = (btail[:, None, :] + pos_tab[:, None, :])
    bpos = jnp.broadcast_to(bpos, (N_SEG, tile_b, D_OUT))

    grid = (bs_pad // tile_b,)
    n_tile = N_SEG * tile_b

    flops = (2 * T_TOT * bs_pad * F_IN * 2 * G
             + 2 * T_TOT * bs_pad * H * 2 * G
             + 2 * N_SEG * bs_pad * 2 * H * D_OUT)
    transcendentals = T_TOT * bs_pad * 2 * G
    bytes_accessed = 4 * T_TOT * bs_pad * F_IN \
        + 4 * (F_IN * 2 * G + H * 2 * G) \
        + 4 * (2 * H * D_OUT + N_SEG * tile_b * D_OUT + N_SEG * bs_pad * D_OUT)

    out_sm = pl.pallas_call(
        _gru_kernel,
        out_shape=jax.ShapeDtypeStruct((N_SEG, bs_pad, D_OUT), jnp.float32),
        grid=grid,
        in_specs=[
            pl.BlockSpec((F_IN, tile_b, T_TOT), lambda i: (0, i, 0)),
            pl.BlockSpec((F_IN, 2 * G), lambda i: (0, 0)),
            pl.BlockSpec((1, 2 * G), lambda i: (0, 0)),
            pl.BlockSpec((H, G), lambda i: (0, 0)),
            pl.BlockSpec((H, G), lambda i: (0, 0)),
            pl.BlockSpec((1, 2 * G), lambda i: (0, 0)),
            pl.BlockSpec((2 * H, D_OUT), lambda i: (0, 0)),
            pl.BlockSpec((N_SEG, tile_b, D_OUT), lambda i: (0, 0, 0)),
        ],
        out_specs=pl.BlockSpec((N_SEG, tile_b, D_OUT), lambda i: (0, i, 0)),
        scratch_shapes=[pltpu.VMEM((T_SEG, n_tile, 2 * G), jnp.bfloat16)],
        compiler_params=pltpu.CompilerParams(
            dimension_semantics=("parallel",),
            vmem_limit_bytes=63 * 1024 * 1024,
        ),
        cost_estimate=pl.CostEstimate(flops=flops,
                                      transcendentals=transcendentals,
                                      bytes_accessed=bytes_accessed),
    )(x, wih, b_fold, whf, whb, bhn, wtail, bpos)

    return jnp.transpose(out_sm[:, :bs, :], (1, 0, 2))
